# baseline (speedup 1.0000x reference)
import functools

import jax
import jax.numpy as jnp
from jax import lax
from jax.experimental import pallas as pl
from jax.experimental.pallas import tpu as pltpu
from jax.experimental.pallas import tpu_sc as plsc

N_PTS = 2_000_000
GRID = 128
CELLS = GRID - 1
TBL = CELLS * CELLS * CELLS * 8
SX = CELLS * CELLS * 8
SY = CELLS * 8
SZ = 8

NW = 32
C = 3200
NCH = N_PTS // C
NB = C // 128

_mesh = plsc.VectorSubcoreMesh(
    core_axis_name="c", subcore_axis_name="s", num_cores=2, num_subcores=16
)


def _tec_body(rx_hbm, ry_hbm, rz_hbm, phi_hbm, tab_hbm, out_hbm,
              rxv, ryv, rzv, phiv, fxv, fyv, fzv, idxv, valsv, outv, gsem):
    wid = lax.axis_index("s") * 2 + lax.axis_index("c")
    nchunks = jnp.where(wid <= NCH % NW - 1 + (NW if NCH % NW == 0 else 0),
                        NCH // NW + 1, NCH // NW)

    def do_chunk(i, carry):
        chunk = wid + i * NW
        base = chunk * C
        pltpu.sync_copy(rx_hbm.at[pl.ds(base, C)], rxv)
        pltpu.sync_copy(ry_hbm.at[pl.ds(base, C)], ryv)
        pltpu.sync_copy(rz_hbm.at[pl.ds(base, C)], rzv)
        pltpu.sync_copy(phi_hbm.at[pl.ds(base, C)], phiv)

        def idx_batch(b, c2):
            for jj in range(8):
                j = b * 8 + jj
                xv = rxv[pl.ds(j * 16, 16)]
                yv = ryv[pl.ds(j * 16, 16)]
                zv = rzv[pl.ds(j * 16, 16)]
                tx = (xv + 1.0) * 63.5
                ty = (yv + 1.0) * 63.5
                tz = (zv + 1.0) * 63.5
                ix = jnp.clip(tx.astype(jnp.int32), 0, GRID - 2)
                iy = jnp.clip(ty.astype(jnp.int32), 0, GRID - 2)
                iz = jnp.clip(tz.astype(jnp.int32), 0, GRID - 2)
                fxv[pl.ds(j * 16, 16)] = jnp.clip(tx - ix.astype(jnp.float32), 0.0, 1.0)
                fyv[pl.ds(j * 16, 16)] = jnp.clip(ty - iy.astype(jnp.float32), 0.0, 1.0)
                fzv[pl.ds(j * 16, 16)] = jnp.clip(tz - iz.astype(jnp.float32), 0.0, 1.0)
                pv = phiv[pl.ds(j * 16, 16)]
                flat = ix * SX + iy * SY + iz * SZ + jnp.where(pv < 0.0, TBL, 0)
                col = jj * 16
                for c in range(8):
                    idxv[c * NB + b, pl.ds(col, 16)] = flat + c
            for c in range(8):
                pltpu.async_copy(tab_hbm.at[idxv.at[c * NB + b]],
                                 valsv.at[c * NB + b], gsem)

            @pl.when(b >= 2)
            def _drain():
                for c in range(8):
                    pltpu.make_async_copy(tab_hbm.at[idxv.at[c * NB + b - 2]],
                                          valsv.at[c * NB + b - 2], gsem).wait()
            return c2

        lax.fori_loop(0, NB, idx_batch, 0)
        for bb in (NB - 2, NB - 1):
            for c in range(8):
                pltpu.make_async_copy(tab_hbm.at[idxv.at[c * NB + bb]],
                                      valsv.at[c * NB + bb], gsem).wait()

        def comb_batch(b, c2):
            for jj in range(8):
                j = b * 8 + jj
                col = jj * 16
                v = [valsv[c * NB + b, pl.ds(col, 16)] for c in range(8)]
                fx = fxv[pl.ds(j * 16, 16)]
                fy = fyv[pl.ds(j * 16, 16)]
                fz = fzv[pl.ds(j * 16, 16)]
                c00 = v[0] * (1.0 - fx) + v[4] * fx
                c01 = v[1] * (1.0 - fx) + v[5] * fx
                c10 = v[2] * (1.0 - fx) + v[6] * fx
                c11 = v[3] * (1.0 - fx) + v[7] * fx
                c0 = c00 * (1.0 - fy) + c10 * fy
                c1 = c01 * (1.0 - fy) + c11 * fy
                outv[pl.ds(j * 16, 16)] = c0 * (1.0 - fz) + c1 * fz
            return c2

        lax.fori_loop(0, NB, comb_batch, 0)
        pltpu.sync_copy(outv, out_hbm.at[pl.ds(base, C)])
        return carry

    lax.fori_loop(0, nchunks, do_chunk, 0)


_interp = functools.partial(
    pl.kernel,
    out_type=jax.ShapeDtypeStruct((N_PTS,), jnp.float32),
    mesh=_mesh,
    scratch_types=[
        pltpu.VMEM((C,), jnp.float32),
        pltpu.VMEM((C,), jnp.float32),
        pltpu.VMEM((C,), jnp.float32),
        pltpu.VMEM((C,), jnp.float32),
        pltpu.VMEM((C,), jnp.float32),
        pltpu.VMEM((C,), jnp.float32),
        pltpu.VMEM((C,), jnp.float32),
        pltpu.VMEM((8 * NB, 128), jnp.int32),
        pltpu.VMEM((8 * NB, 128), jnp.float32),
        pltpu.VMEM((C,), jnp.float32),
        pltpu.SemaphoreType.DMA,
    ],
)(_tec_body)


def _pack_corners(t):
    cs = [t[dx:dx + CELLS, dy:dy + CELLS, dz:dz + CELLS]
          for dx in (0, 1) for dy in (0, 1) for dz in (0, 1)]
    return jnp.stack(cs, axis=-1).reshape(-1)


def kernel(r, phi_r, trainables_m, trainables_p):
    rx, ry, rz = r[:, 0], r[:, 1], r[:, 2]
    tab = jnp.concatenate(
        [_pack_corners(trainables_p), _pack_corners(trainables_m)])
    return _interp(rx, ry, rz, phi_r, tab)


# speedup vs baseline: 3.3462x; 3.3462x over previous
import functools

import jax
import jax.numpy as jnp
from jax import lax
from jax.experimental import pallas as pl
from jax.experimental.pallas import tpu as pltpu
from jax.experimental.pallas import tpu_sc as plsc

N_PTS = 2_000_000
GRID = 128
TBL = GRID * GRID * GRID
SX = GRID * GRID
SY = GRID

NW = 32
C = 3200
NCH = N_PTS // C
NB = C // 128

_mesh = plsc.VectorSubcoreMesh(
    core_axis_name="c", subcore_axis_name="s", num_cores=2, num_subcores=16
)


def _tec_body(rx_hbm, ry_hbm, rz_hbm, phi_hbm, tab_hbm, out_hbm,
              rxv, ryv, rzv, phiv, fxv, fyv, fzv,
              i0, i1, i2, i3, i4, i5, i6, i7,
              v0, v1, v2, v3, v4, v5, v6, v7, outv, gsem):
    idxs = (i0, i1, i2, i3, i4, i5, i6, i7)
    vals = (v0, v1, v2, v3, v4, v5, v6, v7)
    wid = lax.axis_index("s") * 2 + lax.axis_index("c")
    nchunks = jnp.where(wid <= NCH % NW - 1 + (NW if NCH % NW == 0 else 0),
                        NCH // NW + 1, NCH // NW)

    def do_chunk(i, carry):
        chunk = wid + i * NW
        base = chunk * C
        pltpu.sync_copy(rx_hbm.at[pl.ds(base, C)], rxv)
        pltpu.sync_copy(ry_hbm.at[pl.ds(base, C)], ryv)
        pltpu.sync_copy(rz_hbm.at[pl.ds(base, C)], rzv)
        pltpu.sync_copy(phi_hbm.at[pl.ds(base, C)], phiv)

        def idx_step(j, c2):
            xv = rxv[pl.ds(j * 16, 16)]
            yv = ryv[pl.ds(j * 16, 16)]
            zv = rzv[pl.ds(j * 16, 16)]
            tx = (xv + 1.0) * 63.5
            ty = (yv + 1.0) * 63.5
            tz = (zv + 1.0) * 63.5
            ix = jnp.clip(tx.astype(jnp.int32), 0, GRID - 2)
            iy = jnp.clip(ty.astype(jnp.int32), 0, GRID - 2)
            iz = jnp.clip(tz.astype(jnp.int32), 0, GRID - 2)
            fxv[pl.ds(j * 16, 16)] = jnp.clip(tx - ix.astype(jnp.float32), 0.0, 1.0)
            fyv[pl.ds(j * 16, 16)] = jnp.clip(ty - iy.astype(jnp.float32), 0.0, 1.0)
            fzv[pl.ds(j * 16, 16)] = jnp.clip(tz - iz.astype(jnp.float32), 0.0, 1.0)
            pv = phiv[pl.ds(j * 16, 16)]
            flat = ix * SX + iy * SY + iz + jnp.where(pv < 0.0, TBL, 0)
            for c in range(8):
                dx, dy, dz = (c >> 2) & 1, (c >> 1) & 1, c & 1
                idxs[c][pl.ds(j * 16, 16)] = flat + (dx * SX + dy * SY + dz)
            return c2

        lax.fori_loop(0, C // 16, idx_step, 0)
        for c in range(8):
            pltpu.async_copy(tab_hbm.at[idxs[c]], vals[c], gsem)
        for c in range(8):
            pltpu.make_async_copy(tab_hbm.at[idxs[c]], vals[c], gsem).wait()

        def comb_step(j, c2):
            v = [vals[c][pl.ds(j * 16, 16)] for c in range(8)]
            fx = fxv[pl.ds(j * 16, 16)]
            fy = fyv[pl.ds(j * 16, 16)]
            fz = fzv[pl.ds(j * 16, 16)]
            c00 = v[0] * (1.0 - fx) + v[4] * fx
            c01 = v[1] * (1.0 - fx) + v[5] * fx
            c10 = v[2] * (1.0 - fx) + v[6] * fx
            c11 = v[3] * (1.0 - fx) + v[7] * fx
            c0 = c00 * (1.0 - fy) + c10 * fy
            c1 = c01 * (1.0 - fy) + c11 * fy
            outv[pl.ds(j * 16, 16)] = c0 * (1.0 - fz) + c1 * fz
            return c2

        lax.fori_loop(0, C // 16, comb_step, 0)
        pltpu.sync_copy(outv, out_hbm.at[pl.ds(base, C)])
        return carry

    lax.fori_loop(0, nchunks, do_chunk, 0)


_interp = functools.partial(
    pl.kernel,
    out_type=jax.ShapeDtypeStruct((N_PTS,), jnp.float32),
    mesh=_mesh,
    scratch_types=[
        pltpu.VMEM((C,), jnp.float32),
        pltpu.VMEM((C,), jnp.float32),
        pltpu.VMEM((C,), jnp.float32),
        pltpu.VMEM((C,), jnp.float32),
        pltpu.VMEM((C,), jnp.float32),
        pltpu.VMEM((C,), jnp.float32),
        pltpu.VMEM((C,), jnp.float32),
        *[pltpu.VMEM((C,), jnp.int32) for _ in range(8)],
        *[pltpu.VMEM((C,), jnp.float32) for _ in range(8)],
        pltpu.VMEM((C,), jnp.float32),
        pltpu.SemaphoreType.DMA,
    ],
)(_tec_body)


def kernel(r, phi_r, trainables_m, trainables_p):
    rx, ry, rz = r[:, 0], r[:, 1], r[:, 2]
    tab = jnp.concatenate(
        [trainables_p.reshape(-1), trainables_m.reshape(-1)])
    return _interp(rx, ry, rz, phi_r, tab)
